# baseline (speedup 1.0000x reference)
import functools

import jax
import jax.numpy as jnp
from jax import lax
from jax.experimental import pallas as pl
from jax.experimental.pallas import tpu as pltpu
from jax.experimental.pallas import tpu_sc as plsc

_NUM_CORES = 2
_NUM_SUBCORES = 16
_LANES = 16


@functools.partial(jax.jit, static_argnames=("b", "s", "chunk"))
def _embedding_add(idx, W_words, W_pos, b, s, chunk):
    d = W_words.shape[1]
    nw = _NUM_CORES * _NUM_SUBCORES
    seg = s // nw
    splits = seg // chunk
    per_w = b * seg
    grp = b * chunk
    d_vregs = d // _LANES

    mesh = plsc.VectorSubcoreMesh(core_axis_name="c", subcore_axis_name="s")

    @functools.partial(
        pl.kernel,
        out_type=jax.ShapeDtypeStruct((b, s, d), jnp.float32),
        mesh=mesh,
        scratch_types=[
            pltpu.VMEM((per_w,), jnp.int32),
            pltpu.VMEM((2, chunk, d), jnp.float32),
            pltpu.VMEM((2, grp, d), jnp.float32),
            pltpu.SemaphoreType.DMA,
            pltpu.SemaphoreType.DMA,
            pltpu.SemaphoreType.DMA,
            pltpu.SemaphoreType.DMA,
            pltpu.SemaphoreType.DMA,
            pltpu.SemaphoreType.DMA,
            pltpu.SemaphoreType.DMA,
        ],
    )
    def body(
        idx_hbm, words_hbm, pos_hbm, out_hbm,
        idx_v, pos_v, rows_v, g0, g1, p0, p1, o0, o1, isem,
    ):
        gsem = (g0, g1)
        psem = (p0, p1)
        osem = (o0, o1)
        wid = lax.axis_index("s") * _NUM_CORES + lax.axis_index("c")
        s0 = wid * seg
        icp = pltpu.async_copy(idx_hbm.at[pl.ds(wid * per_w, per_w)], idx_v, isem)

        def fetch(g, sl):
            pltpu.async_copy(
                pos_hbm.at[pl.ds(s0 + g * chunk, chunk)], pos_v.at[sl], psem[sl]
            )
            pltpu.async_copy(
                words_hbm.at[idx_v.at[pl.ds(pl.multiple_of(g * grp, grp), grp)]],
                rows_v.at[sl],
                gsem[sl],
            )

        def wait_fetch(sl):
            pltpu.make_async_copy(
                pos_hbm.at[pl.ds(s0, chunk)], pos_v.at[sl], psem[sl]
            ).wait()
            pltpu.make_async_copy(
                words_hbm.at[idx_v.at[pl.ds(0, grp)]], rows_v.at[sl], gsem[sl]
            ).wait()

        def drain_out(sl):
            for bi in range(b):
                pltpu.make_async_copy(
                    rows_v.at[sl, pl.ds(bi * chunk, chunk)],
                    out_hbm.at[bi, pl.ds(s0, chunk)],
                    osem[sl],
                ).wait()

        def push_out(g, sl):
            for bi in range(b):
                pltpu.async_copy(
                    rows_v.at[sl, pl.ds(bi * chunk, chunk)],
                    out_hbm.at[bi, pl.ds(s0 + g * chunk, chunk)],
                    osem[sl],
                )

        def process(g, sl):
            wait_fetch(sl)

            @plsc.parallel_loop(0, chunk, 1, unroll=2)
            def add(r, _sl=sl):
                for jv in range(d_vregs):
                    j = jv * _LANES
                    pv = pos_v[_sl, r, pl.ds(j, _LANES)]
                    for bi in range(b):
                        plsc.addupdate(
                            rows_v.at[_sl, bi * chunk + r, pl.ds(j, _LANES)], pv
                        )

            push_out(g, sl)

        icp.wait()
        fetch(0, 0)

        def round_body(k, _):
            g0 = 2 * k

            @pl.when(k >= 1)
            def _():
                drain_out(1)

            fetch(g0 + 1, 1)
            process(g0, 0)

            @pl.when(k + 1 < splits // 2)
            def _():
                drain_out(0)
                fetch(g0 + 2, 0)

            process(g0 + 1, 1)
            return 0

        lax.fori_loop(0, splits // 2, round_body, 0)
        drain_out(0)
        drain_out(1)

    return body(idx, W_words, W_pos)


def kernel(input_ids, W_words, W_pos):
    b, s = input_ids.shape
    nw = _NUM_CORES * _NUM_SUBCORES
    chunk = 8
    seg = s // nw
    splits = seg // chunk
    idx = jnp.transpose(
        input_ids.astype(jnp.int32).reshape(b, nw, splits, chunk), (1, 2, 0, 3)
    ).reshape(b * s)
    return _embedding_add(idx, W_words, W_pos, b=b, s=s, chunk=chunk)


# speedup vs baseline: 1.0959x; 1.0959x over previous
import functools

import jax
import jax.numpy as jnp
from jax import lax
from jax.experimental import pallas as pl
from jax.experimental.pallas import tpu as pltpu
from jax.experimental.pallas import tpu_sc as plsc

_NUM_CORES = 2
_NUM_SUBCORES = 16
_LANES = 16


@functools.partial(jax.jit, static_argnames=("b", "s", "chunk"))
def _embedding_add(idx, W_words, W_pos, b, s, chunk):
    d = W_words.shape[1]
    nw = _NUM_CORES * _NUM_SUBCORES
    seg = s // nw
    splits = seg // chunk
    per_w = b * seg
    grp = b * chunk
    d_vregs = d // _LANES

    mesh = plsc.VectorSubcoreMesh(core_axis_name="c", subcore_axis_name="s")

    @functools.partial(
        pl.kernel,
        out_type=jax.ShapeDtypeStruct((b, s, d), jnp.float32),
        mesh=mesh,
        scratch_types=[
            pltpu.VMEM((per_w,), jnp.int32),
            pltpu.VMEM((2, chunk, d), jnp.float32),
            pltpu.VMEM((2, grp, d), jnp.float32),
            pltpu.SemaphoreType.DMA,
            pltpu.SemaphoreType.DMA,
            pltpu.SemaphoreType.DMA,
            pltpu.SemaphoreType.DMA,
            pltpu.SemaphoreType.DMA,
            pltpu.SemaphoreType.DMA,
            pltpu.SemaphoreType.DMA,
        ],
    )
    def body(
        idx_hbm, words_hbm, pos_hbm, out_hbm,
        idx_v, pos_v, rows_v, g0, g1, p0, p1, o0, o1, isem,
    ):
        gsem = (g0, g1)
        psem = (p0, p1)
        osem = (o0, o1)
        wid = lax.axis_index("s") * _NUM_CORES + lax.axis_index("c")
        s0 = wid * seg
        icp = pltpu.async_copy(idx_hbm.at[pl.ds(wid * per_w, per_w)], idx_v, isem)

        def fetch(g, sl):
            pltpu.async_copy(
                pos_hbm.at[pl.ds(s0 + g * chunk, chunk)], pos_v.at[sl], psem[sl]
            )
            pltpu.async_copy(
                words_hbm.at[idx_v.at[pl.ds(pl.multiple_of(g * grp, grp), grp)]],
                rows_v.at[sl],
                gsem[sl],
            )

        def wait_fetch(sl):
            pltpu.make_async_copy(
                pos_hbm.at[pl.ds(s0, chunk)], pos_v.at[sl], psem[sl]
            ).wait()
            pltpu.make_async_copy(
                words_hbm.at[idx_v.at[pl.ds(0, grp)]], rows_v.at[sl], gsem[sl]
            ).wait()

        def drain_out(sl):
            for bi in range(b):
                pltpu.make_async_copy(
                    rows_v.at[sl, pl.ds(bi * chunk, chunk)],
                    out_hbm.at[bi, pl.ds(s0, chunk)],
                    osem[sl],
                ).wait()

        def push_out(g, sl):
            for bi in range(b):
                pltpu.async_copy(
                    rows_v.at[sl, pl.ds(bi * chunk, chunk)],
                    out_hbm.at[bi, pl.ds(s0 + g * chunk, chunk)],
                    osem[sl],
                )

        def process(g, sl):
            wait_fetch(sl)

            @plsc.parallel_loop(0, chunk, 1, unroll=1)
            def add(r, _sl=sl):
                for jv in range(d_vregs):
                    j = jv * _LANES
                    pv = pos_v[_sl, r, pl.ds(j, _LANES)]
                    for bi in range(b):
                        plsc.addupdate(
                            rows_v.at[_sl, bi * chunk + r, pl.ds(j, _LANES)], pv
                        )

            push_out(g, sl)

        icp.wait()
        fetch(0, 0)

        def round_body(k, _):
            g0 = 2 * k

            @pl.when(k >= 1)
            def _():
                drain_out(1)

            fetch(g0 + 1, 1)
            process(g0, 0)

            @pl.when(k + 1 < splits // 2)
            def _():
                drain_out(0)
                fetch(g0 + 2, 0)

            process(g0 + 1, 1)
            return 0

        lax.fori_loop(0, splits // 2, round_body, 0)
        drain_out(0)
        drain_out(1)

    return body(idx, W_words, W_pos)


def kernel(input_ids, W_words, W_pos):
    b, s = input_ids.shape
    nw = _NUM_CORES * _NUM_SUBCORES
    chunk = 8
    seg = s // nw
    splits = seg // chunk
    idx = jnp.transpose(
        input_ids.astype(jnp.int32).reshape(b, nw, splits, chunk), (1, 2, 0, 3)
    ).reshape(b * s)
    return _embedding_add(idx, W_words, W_pos, b=b, s=s, chunk=chunk)
